# baseline (speedup 1.0000x reference)
import functools

import jax
import jax.numpy as jnp
from jax import lax
from jax.experimental import pallas as pl
from jax.experimental.pallas import tpu as pltpu
from jax.experimental.pallas import tpu_sc as plsc

N = 10000
E = 320000
D = 128

NC = 2
NS = 16
NW = NC * NS

CHUNK = 128
CPW = 80
E_PAD = NW * CPW * CHUNK
NACC = N + 8

RPS = 632
RPS_LAST = N - RPS * (NS - 1)


def _sc_segment_sum(feat, srcp, dstp):
    mesh = plsc.VectorSubcoreMesh(core_axis_name="c", subcore_axis_name="s")

    @functools.partial(
        pl.kernel,
        out_type=jax.ShapeDtypeStruct((NC * N, D), jnp.float32),
        mesh=mesh,
        scratch_types=[
            pltpu.VMEM((CPW // 2, CHUNK), jnp.int32),
            pltpu.VMEM((CPW // 2, CHUNK), jnp.int32),
            pltpu.VMEM((CHUNK, D), jnp.float32),
            pltpu.VMEM((CHUNK, D), jnp.float32),
            pltpu.VMEM_SHARED((NACC, D), jnp.float32),
            pltpu.SemaphoreType.DMA,
            pltpu.SemaphoreType.DMA,
            pltpu.SemaphoreType.DMA,
            pltpu.SemaphoreType.DMA,
        ],
    )
    def k(feat_hbm, src_hbm, dst_hbm, out_hbm,
          sidx, didx, rows0, rows1, acc, g0, g1, s0, s1):
        c = lax.axis_index("c")
        s = lax.axis_index("s")
        wid = c * NS + s

        @pl.loop(0, CHUNK)
        def _(i):
            @pl.loop(0, D, step=16)
            def _(j):
                rows0.at[i, pl.ds(j, 16)][...] = jnp.zeros((16,), jnp.float32)

        base_r = s * RPS

        def zero_rows(tail):
            @pl.loop(0, 4)
            def _(r):
                pltpu.sync_copy(rows0, acc.at[pl.ds(base_r + r * CHUNK, CHUNK)])
            pltpu.sync_copy(rows0.at[pl.ds(0, tail)],
                            acc.at[pl.ds(base_r + 4 * CHUNK, tail)])

        @pl.when(s < NS - 1)
        def _():
            zero_rows(RPS - 4 * CHUNK)

        @pl.when(s == NS - 1)
        def _():
            zero_rows(RPS_LAST + 8 - 4 * CHUNK)

        plsc.subcore_barrier()

        HALF = CPW // 2
        for h in range(2):
            base = wid * CPW + h * HALF
            pltpu.sync_copy(src_hbm.at[pl.ds(base, HALF)], sidx)
            pltpu.sync_copy(dst_hbm.at[pl.ds(base, HALF)], didx)

            pltpu.async_copy(feat_hbm.at[sidx.at[0]], rows0, g0)
            pltpu.async_copy(feat_hbm.at[sidx.at[1]], rows1, g1)

            @pl.loop(0, HALF, step=2)
            def _(t):
                pltpu.make_async_copy(feat_hbm.at[sidx.at[t]], rows0, g0).wait()
                pltpu.async_copy(rows0, acc.at[didx.at[t]], s0, add=True)
                pltpu.make_async_copy(feat_hbm.at[sidx.at[t + 1]], rows1,
                                      g1).wait()
                pltpu.async_copy(rows1, acc.at[didx.at[t + 1]], s1, add=True)

                pltpu.make_async_copy(rows0, acc.at[didx.at[t]], s0).wait()

                @pl.when(t + 2 < HALF)
                def _():
                    pltpu.async_copy(feat_hbm.at[sidx.at[t + 2]], rows0, g0)

                pltpu.make_async_copy(rows1, acc.at[didx.at[t + 1]], s1).wait()

                @pl.when(t + 3 < HALF)
                def _():
                    pltpu.async_copy(feat_hbm.at[sidx.at[t + 3]], rows1, g1)

        plsc.subcore_barrier()

        @pl.when(s < NS - 1)
        def _():
            pltpu.sync_copy(acc.at[pl.ds(base_r, RPS)],
                            out_hbm.at[pl.ds(c * N + base_r, RPS)])

        @pl.when(s == NS - 1)
        def _():
            pltpu.sync_copy(acc.at[pl.ds(base_r, RPS_LAST)],
                            out_hbm.at[pl.ds(c * N + base_r, RPS_LAST)])

    return k(feat, srcp, dstp)


def _tc_layer(x, p0, p1, W, b2d, final):
    BR = 1000

    def body(x_ref, p0_ref, p1_ref, w_ref, b_ref, o_ref):
        t = x_ref[...] + p0_ref[...] + p1_ref[...]
        acc = jnp.dot(t, w_ref[...], preferred_element_type=jnp.float32,
                      precision=lax.Precision.HIGHEST) + b_ref[...]
        if final:
            m = jnp.max(acc, axis=1, keepdims=True)
            e = acc - m
            lse = jnp.log(jnp.sum(jnp.exp(e), axis=1, keepdims=True))
            o_ref[...] = e - lse
        else:
            o_ref[...] = jnp.maximum(acc, 0.0)

    return pl.pallas_call(
        body,
        grid=(N // BR,),
        in_specs=[
            pl.BlockSpec((BR, D), lambda i: (i, 0)),
            pl.BlockSpec((BR, D), lambda i: (i, 0)),
            pl.BlockSpec((BR, D), lambda i: (i, 0)),
            pl.BlockSpec((D, D), lambda i: (0, 0)),
            pl.BlockSpec((1, D), lambda i: (0, 0)),
        ],
        out_specs=pl.BlockSpec((BR, D), lambda i: (i, 0)),
        out_shape=jax.ShapeDtypeStruct((N, D), jnp.float32),
    )(x, p0, p1, W, b2d)


def kernel(input_feature, edge_index, W1, b1, W2, b2):
    src = edge_index[0]
    dst = edge_index[1]
    npad = E_PAD - E
    srcp = jnp.concatenate([src, jnp.zeros((npad,), jnp.int32)])
    dstp = jnp.concatenate([dst, jnp.full((npad,), N, jnp.int32)])
    srcp = srcp.reshape(NW * CPW, CHUNK)
    dstp = dstp.reshape(NW * CPW, CHUNK)
    b1_2d = b1.reshape(1, D)
    b2_2d = b2.reshape(1, D)

    p = _sc_segment_sum(input_feature, srcp, dstp)
    h = _tc_layer(input_feature, p[:N], p[N:], W1, b1_2d, final=False)
    q = _sc_segment_sum(h, srcp, dstp)
    return _tc_layer(h, q[:N], q[N:], W2, b2_2d, final=True)
